# Initial kernel scaffold; baseline (speedup 1.0000x reference)
import jax
import jax.numpy as jnp
from jax.experimental import pallas as pl


def kernel(x, edge_index, batch, W1, b1, W2, b2, W3, b3):
    raise NotImplementedError("write your pallas kernel here")


# speedup vs baseline: 27.7284x; 27.7284x over previous
import functools

import jax
import jax.numpy as jnp
from jax import lax
from jax.experimental import pallas as pl
from jax.experimental.pallas import tpu as pltpu
from jax.experimental.pallas import tpu_sc as plsc

N = 10000
E = 320000
F = 128
H = 32
G = 64

NC = 2
NS = 16
NW = NC * NS
L = 16

NPAD = 10240
RPW = NPAD // NS
CH = 128
NCHUNK = -(-E // (NW * CH))
EPW = NCHUNK * CH
EP = EPW * NW

_MESH = plsc.VectorSubcoreMesh(core_axis_name="c", subcore_axis_name="s")
_SC_PARAMS = pltpu.CompilerParams(needs_layout_passes=False,
                                  use_tc_tiling_on_sc=False)


@functools.partial(
    pl.kernel,
    out_type=jax.ShapeDtypeStruct((NC, NPAD), jnp.float32),
    mesh=_MESH,
    scratch_types=[
        pltpu.VMEM((NCHUNK, CH), jnp.int32),
        pltpu.VMEM((NPAD,), jnp.float32),
        pltpu.VMEM((NS, RPW), jnp.float32),
        pltpu.VMEM_SHARED((NS, NPAD), jnp.float32),
    ],
    compiler_params=_SC_PARAMS,
)
def _deg_kernel(dst_hbm, out_hbm, idx_v, hist_v, blk_v, shared):
    c = lax.axis_index("c")
    s = lax.axis_index("s")
    wid = s * NC + c
    zeros16 = jnp.zeros((L,), jnp.float32)
    ones16 = jnp.ones((L,), jnp.float32)

    def zero_body(i, carry):
        hist_v[pl.ds(i * L, L)] = zeros16
        return carry

    lax.fori_loop(0, NPAD // L, zero_body, 0)
    pltpu.sync_copy(dst_hbm.at[wid], idx_v)

    def hist_body(k, carry):
        for j in range(CH // L):
            idx = idx_v[k, pl.ds(j * L, L)]
            plsc.addupdate_scatter(hist_v, [idx], ones16)
        return carry

    lax.fori_loop(0, NCHUNK, hist_body, 0)
    pltpu.sync_copy(hist_v, shared.at[s])
    plsc.subcore_barrier()

    pltpu.sync_copy(shared.at[:, pl.ds(s * RPW, RPW)], blk_v)

    def red_body(j, carry):
        v = blk_v[0, pl.ds(j * L, L)]
        for r in range(1, NS):
            v = v + blk_v[r, pl.ds(j * L, L)]
        hist_v[pl.ds(j * L, L)] = v
        return carry

    lax.fori_loop(0, RPW // L, red_body, 0)
    pltpu.sync_copy(hist_v.at[pl.ds(0, RPW)], out_hbm.at[c, pl.ds(s * RPW, RPW)])


@functools.partial(
    pl.kernel,
    out_type=jax.ShapeDtypeStruct((NC, NPAD, H), jnp.float32),
    mesh=_MESH,
    scratch_types=[
        pltpu.VMEM((NCHUNK, CH), jnp.int32),
        pltpu.VMEM((NCHUNK, CH), jnp.int32),
        pltpu.VMEM((CH, H), jnp.float32),
        pltpu.VMEM((RPW, H), jnp.float32),
        pltpu.VMEM_SHARED((NPAD, H), jnp.float32),
        pltpu.SemaphoreType.DMA,
    ],
    compiler_params=_SC_PARAMS,
)
def _agg_kernel(p_hbm, src_hbm, dst_hbm, out_hbm,
                src_v, dst_v, rows_v, wb_v, shared, sem):
    c = lax.axis_index("c")
    s = lax.axis_index("s")
    wid = s * NC + c
    zeros16 = jnp.zeros((L,), jnp.float32)

    def zero_body(i, carry):
        for k in range(H // L):
            wb_v[i, pl.ds(k * L, L)] = zeros16
        return carry

    lax.fori_loop(0, RPW, zero_body, 0)
    pltpu.sync_copy(wb_v, shared.at[pl.ds(s * RPW, RPW)])
    pltpu.sync_copy(src_hbm.at[wid], src_v)
    pltpu.sync_copy(dst_hbm.at[wid], dst_v)
    plsc.subcore_barrier()

    def chunk_body(k, carry):
        pltpu.async_copy(p_hbm.at[src_v.at[k]], rows_v, sem).wait()
        pltpu.sync_copy(rows_v, shared.at[dst_v.at[k]], add=True)
        return carry

    lax.fori_loop(0, NCHUNK, chunk_body, 0)
    plsc.subcore_barrier()
    pltpu.sync_copy(shared.at[pl.ds(s * RPW, RPW)], wb_v)
    pltpu.sync_copy(wb_v, out_hbm.at[c, pl.ds(s * RPW, RPW)])


def _tc1_body(x_ref, w1_ref, degt_ref, p1_ref, dinv_ref):
    deg = degt_ref[:, 0:1] + degt_ref[:, 1:2] + 1.0
    dinv = lax.rsqrt(deg)
    dinv_ref[...] = dinv
    u = jnp.dot(x_ref[...], w1_ref[...], preferred_element_type=jnp.float32)
    p1_ref[...] = u * dinv


_tc1 = pl.pallas_call(
    _tc1_body,
    out_shape=(jax.ShapeDtypeStruct((NPAD, H), jnp.float32),
               jax.ShapeDtypeStruct((NPAD, 1), jnp.float32)),
)


def _tc2_body(aggp_ref, p1_ref, dinv_ref, b1_ref, w2_ref, p2_ref):
    agg = aggp_ref[0] + aggp_ref[1] + p1_ref[...]
    h1 = jnp.maximum(dinv_ref[...] * agg + b1_ref[...], 0.0)
    p2_ref[...] = jnp.dot(h1, w2_ref[...],
                          preferred_element_type=jnp.float32) * dinv_ref[...]


_tc2 = pl.pallas_call(
    _tc2_body,
    out_shape=jax.ShapeDtypeStruct((NPAD, H), jnp.float32),
)


def _tc3_body(aggp_ref, p2_ref, dinv_ref, b2_ref, batch_ref, w3_ref, b3_ref,
              out_ref):
    agg = aggp_ref[0] + aggp_ref[1] + p2_ref[...]
    h2 = jnp.maximum(dinv_ref[...] * agg + b2_ref[...], 0.0)
    gids = lax.broadcasted_iota(jnp.int32, (G, NPAD), 0)
    onehot = (batch_ref[...] == gids).astype(jnp.float32)
    sums = jnp.dot(onehot, h2, preferred_element_type=jnp.float32)
    counts = jnp.sum(onehot, axis=1, keepdims=True)
    pooled = sums / jnp.maximum(counts, 1.0)
    out_ref[...] = jnp.dot(pooled, w3_ref[...],
                           preferred_element_type=jnp.float32) + b3_ref[...]


_tc3 = pl.pallas_call(
    _tc3_body,
    out_shape=jax.ShapeDtypeStruct((G, 1), jnp.float32),
)


def kernel(x, edge_index, batch, W1, b1, W2, b2, W3, b3):
    src = edge_index[0].astype(jnp.int32)
    dst = edge_index[1].astype(jnp.int32)
    src3 = jnp.pad(src, (0, EP - E), constant_values=N).reshape(NW, NCHUNK, CH)
    dst3 = jnp.pad(dst, (0, EP - E), constant_values=N).reshape(NW, NCHUNK, CH)
    x_pad = jnp.pad(x, ((0, NPAD - N), (0, 0)))
    batch_pad = jnp.pad(batch.astype(jnp.int32), (0, NPAD - N),
                        constant_values=G).reshape(1, NPAD)

    degp = _deg_kernel(dst3)
    degt = degp.T
    p1, dinv = _tc1(x_pad, W1, degt)
    agg1 = _agg_kernel(p1, src3, dst3)
    p2 = _tc2(agg1, p1, dinv, b1.reshape(1, H), W2)
    agg2 = _agg_kernel(p2, src3, dst3)
    out = _tc3(agg2, p2, dinv, b2.reshape(1, H), batch_pad, W3,
               b3.reshape(1, 1))
    return out.reshape(-1)
